# baseline (speedup 1.0000x reference)
import functools

import jax
import jax.numpy as jnp
from jax import lax
from jax.experimental import pallas as pl
from jax.experimental.pallas import tpu as pltpu
from jax.experimental.pallas import tpu_sc as plsc


def _emb_call(b, s, d, x_t, table_p):
    info = plsc.get_sparse_core_info()
    nw = info.num_cores * info.num_subcores
    n_units = (s // 8) * (b // 128)
    per_w = n_units // nw
    nb = b // 128

    mesh = plsc.VectorSubcoreMesh(core_axis_name="c", subcore_axis_name="s")

    @functools.partial(
        pl.kernel,
        mesh=mesh,
        out_type=jax.ShapeDtypeStruct((s, d, b), jnp.float32),
        scratch_types=[
            pltpu.VMEM((2, 8, 128), jnp.int32),
            pltpu.VMEM((2, 128, 128), jnp.float32),
            pltpu.VMEM((2, 64, 128), jnp.float32),
            pltpu.SemaphoreType.DMA((2,)),
            pltpu.SemaphoreType.DMA((2,)),
        ],
        compiler_params=pltpu.CompilerParams(needs_layout_passes=False),
    )
    def emb(x_hbm, table_hbm, out_hbm, xblk, gbuf, tbuf, gsem, wsem):
        wid = lax.axis_index("s") * info.num_cores + lax.axis_index("c")
        u0 = wid * per_w
        iota = lax.iota(jnp.int32, 16)

        def load_xblk(u, p):
            so = u // nb
            bb = u % nb
            pltpu.sync_copy(
                x_hbm.at[pl.ds(so * 8, 8), pl.ds(bb * 128, 128)], xblk.at[p])

        def g_start(c, gb):
            r, p = c % 8, (c // 8) % 2
            pltpu.async_copy(
                table_hbm.at[xblk.at[p].at[r]], gbuf.at[gb], gsem.at[gb])

        def g_wait(c, gb):
            r, p = c % 8, (c // 8) % 2
            pltpu.make_async_copy(
                table_hbm.at[xblk.at[p].at[r]], gbuf.at[gb], gsem.at[gb]).wait()

        def w_start(c, gb):
            u, r = u0 + c // 8, c % 8
            so = u // nb
            bb = u % nb
            pltpu.async_copy(
                tbuf.at[gb],
                out_hbm.at[so * 8 + r].at[:, pl.ds(bb * 128, 128)],
                wsem.at[gb])

        def w_wait(gb):
            pltpu.make_async_copy(
                tbuf.at[gb], out_hbm.at[0].at[:, pl.ds(0, 128)],
                wsem.at[gb]).wait()

        def transpose(gb):
            g2 = gbuf.at[gb]
            t2 = tbuf.at[gb]

            def col(dcol, carry):
                cidx = jnp.full((16,), dcol, jnp.int32)
                for g in range(8):
                    val = plsc.load_gather(g2, [iota + (g * 16), cidx])
                    t2[dcol, pl.ds(g * 16, 16)] = val
                return carry

            lax.fori_loop(0, 64, col, 0)

        n_ch = per_w * 8

        def prefetch(c):
            nc = c + 1

            @pl.when((nc % 8 == 0) & (nc < n_ch))
            def _():
                load_xblk(u0 + nc // 8, (nc // 8) % 2)

            @pl.when(nc < n_ch)
            def _():
                g_start(nc, nc % 2)

        load_xblk(u0, 0)
        g_start(0, 0)
        for c in (0, 1):
            gb = c % 2
            g_wait(c, gb)
            prefetch(c)
            transpose(gb)
            w_start(c, gb)

        def body(t, carry):
            for gb in range(2):
                c = 2 * t + 2 + gb
                g_wait(c, gb)
                prefetch(c)
                w_wait(gb)
                transpose(gb)
                w_start(c, gb)
            return carry

        lax.fori_loop(0, (n_ch - 2) // 2, body, 0)
        w_wait(0)
        w_wait(1)

    return emb(x_t, table_p)


def kernel(x, table):
    b, s = x.shape
    v, d = table.shape
    x_t = x.T.astype(jnp.int32)
    table_p = jnp.pad(table, ((0, 0), (0, 128 - d)))
    p = _emb_call(b, s, d, x_t, table_p)
    return jnp.transpose(p, (2, 0, 1))


# speedup vs baseline: 1.1479x; 1.1479x over previous
import functools

import jax
import jax.numpy as jnp
from jax import lax
from jax.experimental import pallas as pl
from jax.experimental.pallas import tpu as pltpu
from jax.experimental.pallas import tpu_sc as plsc


def _emb_call(b, s, d, x_t, table_p):
    info = plsc.get_sparse_core_info()
    nw = info.num_cores * info.num_subcores
    n_units = (s // 8) * (b // 128)
    per_w = n_units // nw
    nb = b // 128

    mesh = plsc.VectorSubcoreMesh(core_axis_name="c", subcore_axis_name="s")

    @functools.partial(
        pl.kernel,
        mesh=mesh,
        out_type=jax.ShapeDtypeStruct((s, d, b), jnp.float32),
        scratch_types=[
            pltpu.VMEM((2, 8, 128), jnp.int32),
            pltpu.VMEM((2, 128, 128), jnp.float32),
            pltpu.VMEM((2, 64, 129), jnp.float32),
            pltpu.SemaphoreType.DMA((2,)),
            pltpu.SemaphoreType.DMA((2,)),
        ],
        compiler_params=pltpu.CompilerParams(needs_layout_passes=False),
    )
    def emb(x_hbm, table_hbm, out_hbm, xblk, gbuf, tbuf, gsem, wsem):
        wid = lax.axis_index("s") * info.num_cores + lax.axis_index("c")
        u0 = wid * per_w
        iota = lax.iota(jnp.int32, 16)

        def load_xblk(u, p):
            so = u // nb
            bb = u % nb
            pltpu.sync_copy(
                x_hbm.at[pl.ds(so * 8, 8), pl.ds(bb * 128, 128)], xblk.at[p])

        def g_start(c, gb):
            r, p = c % 8, (c // 8) % 2
            pltpu.async_copy(
                table_hbm.at[xblk.at[p].at[r]], gbuf.at[gb], gsem.at[gb])

        def g_wait(c, gb):
            r, p = c % 8, (c // 8) % 2
            pltpu.make_async_copy(
                table_hbm.at[xblk.at[p].at[r]], gbuf.at[gb], gsem.at[gb]).wait()

        def w_start(c, gb):
            u, r = u0 + c // 8, c % 8
            so = u // nb
            bb = u % nb
            pltpu.async_copy(
                tbuf.at[gb].at[:, pl.ds(0, 128)],
                out_hbm.at[so * 8 + r].at[:, pl.ds(bb * 128, 128)],
                wsem.at[gb])

        def w_wait(gb):
            pltpu.make_async_copy(
                tbuf.at[gb].at[:, pl.ds(0, 128)],
                out_hbm.at[0].at[:, pl.ds(0, 128)],
                wsem.at[gb]).wait()

        kiota = [iota + (16 * k) for k in range(4)]

        def transpose(gb):
            g2 = gbuf.at[gb]
            t2 = tbuf.at[gb]

            def tok(t, carry):
                tcol = jnp.full((16,), t, jnp.int32)
                for k in range(4):
                    v = g2[t, pl.ds(k * 16, 16)]
                    plsc.store_scatter(t2, [kiota[k], tcol], v)
                return carry

            lax.fori_loop(0, 128, tok, 0)

        n_ch = per_w * 8

        def prefetch(c):
            nc = c + 1

            @pl.when((nc % 8 == 0) & (nc < n_ch))
            def _():
                load_xblk(u0 + nc // 8, (nc // 8) % 2)

            @pl.when(nc < n_ch)
            def _():
                g_start(nc, nc % 2)

        load_xblk(u0, 0)
        g_start(0, 0)
        for c in (0, 1):
            gb = c % 2
            g_wait(c, gb)
            prefetch(c)
            transpose(gb)
            w_start(c, gb)

        def body(t, carry):
            for gb in range(2):
                c = 2 * t + 2 + gb
                g_wait(c, gb)
                prefetch(c)
                w_wait(gb)
                transpose(gb)
                w_start(c, gb)
            return carry

        lax.fori_loop(0, (n_ch - 2) // 2, body, 0)
        w_wait(0)
        w_wait(1)

    return emb(x_t, table_p)


def kernel(x, table):
    b, s = x.shape
    v, d = table.shape
    x_t = x.T.astype(jnp.int32)
    table_p = jnp.pad(table, ((0, 0), (0, 128 - d)))
    p = _emb_call(b, s, d, x_t, table_p)
    return jnp.transpose(p, (2, 0, 1))
